# Initial kernel scaffold; baseline (speedup 1.0000x reference)
import jax
import jax.numpy as jnp
from jax.experimental import pallas as pl


def kernel(x):
    raise NotImplementedError("write your pallas kernel here")


# speedup vs baseline: 6.6491x; 6.6491x over previous
import functools

import jax
import jax.numpy as jnp
from jax import lax
from jax.experimental import pallas as pl
from jax.experimental.pallas import tpu as pltpu
from jax.experimental.pallas import tpu_sc as plsc

K = 1024
N = 32768
R = 1024
NV = N // 16
NC, NS = 2, 16
NW = NC * NS
RPW = R // NW

_mesh = plsc.VectorSubcoreMesh(
    core_axis_name="c", subcore_axis_name="s", num_cores=NC, num_subcores=NS
)


_SCRATCH = [
    pltpu.VMEM((N + 16,), jnp.float32),
    pltpu.VMEM((N + 16,), jnp.int32),
    pltpu.VMEM((4096,), jnp.int32),
    pltpu.VMEM((K + 16,), jnp.int32),
    pltpu.VMEM((K,), jnp.float32),
]


def _kmax_body(x_hbm, out_hbm, row_v, tie_v, hist_v, cand_v, orow_v):
    wid = lax.axis_index("s") * NC + lax.axis_index("c")
    lane = lax.iota(jnp.int32, 16)
    ones = jnp.ones((16,), jnp.int32)
    zeros16 = jnp.zeros((16,), jnp.int32)

    def mono16(vf):
        b = plsc.bitcast(vf, jnp.int32)
        return b ^ lax.shift_right_logical(lax.shift_right_arithmetic(b, 31), 1)

    def clear_hist():
        def clr(i, c):
            hist_v[pl.ds(i * 16, 16)] = zeros16
            return c
        lax.fori_loop(0, 256, clr, 0)

    def find_bstar(need):
        def cond(c):
            _, acc, _ = c
            return acc < need
        def body(c):
            b, acc, _ = c
            b2 = b - 1
            cnt = jnp.sum(hist_v[pl.ds(b2 * 16, 16)])
            return (b2, acc + cnt, cnt)
        b, acc, last = lax.while_loop(
            cond, body, (jnp.int32(256), jnp.int32(0), jnp.int32(0))
        )
        return b, acc - last, last

    def do_row(j, carry):
        r = wid * RPW + j
        pltpu.sync_copy(x_hbm.at[r], row_v.at[pl.ds(0, N)])

        clear_hist()

        def h1(i, c):
            k = mono16(row_v[pl.ds(i * 16, 16)])
            b = lax.shift_right_arithmetic(k, 24) + 128
            plsc.addupdate_scatter(hist_v, [b * 16 + lane], ones)
            return c
        lax.fori_loop(0, NV, h1, 0)

        b1, g1, n2 = find_bstar(jnp.int32(K))

        zv = jnp.zeros((16,), jnp.int32)

        def c1(i, carry):
            cov, tov = carry
            k = mono16(row_v[pl.ds(i * 16, 16)])
            b = lax.shift_right_arithmetic(k, 24) + 128
            mgt = b > b1
            meq = b == b1
            gi = mgt.astype(jnp.int32)
            ei = meq.astype(jnp.int32)
            plsc.store_scatter(cand_v, [cov + plsc.cumsum(gi) - gi], k, mask=mgt)
            plsc.store_scatter(tie_v, [tov + plsc.cumsum(ei) - ei], k, mask=meq)
            return (cov + plsc.all_reduce_population_count(mgt),
                    tov + plsc.all_reduce_population_count(meq))
        lax.fori_loop(0, NV, c1, (zv, zv))
        cand_off = g1
        need = jnp.int32(K) - g1

        def level(load_fn, store_fn, shift, n, need, cand_off):
            clear_hist()
            nv = lax.shift_right_logical(n + 15, 4)

            def h(i, c):
                k = load_fn(i)
                valid = (i * 16 + lane) < n
                b = jnp.bitwise_and(lax.shift_right_logical(k, shift), 0xFF)
                plsc.addupdate_scatter(hist_v, [b * 16 + lane], ones, mask=valid)
                return c
            lax.fori_loop(0, nv, h, 0)

            bs, g, n_next = find_bstar(need)
            zv = jnp.zeros((16,), jnp.int32)
            cov0 = jnp.broadcast_to(cand_off, (16,)).astype(jnp.int32)

            def c(i, carry):
                cov, tov = carry
                k = load_fn(i)
                valid = (i * 16 + lane) < n
                b = jnp.bitwise_and(lax.shift_right_logical(k, shift), 0xFF)
                mgt = (b > bs) & valid
                gi = mgt.astype(jnp.int32)
                plsc.store_scatter(cand_v, [cov + plsc.cumsum(gi) - gi], k, mask=mgt)
                cov = cov + plsc.all_reduce_population_count(mgt)
                if store_fn is not None:
                    meq = (b == bs) & valid
                    ei = meq.astype(jnp.int32)
                    store_fn(tov + plsc.cumsum(ei) - ei, k, meq)
                    tov = tov + plsc.all_reduce_population_count(meq)
                return (cov, tov)
            lax.fori_loop(0, nv, c, (cov0, zv))
            return bs, cand_off + g, n_next, need - g

        def load_tie(i):
            return tie_v[pl.ds(i * 16, 16)]

        def load_row_bits(i):
            return plsc.bitcast(row_v[pl.ds(i * 16, 16)], jnp.int32)

        def store_row_bits(idx, k, m):
            plsc.store_scatter(row_v, [idx], plsc.bitcast(k, jnp.float32), mask=m)

        def store_tie(idx, k, m):
            plsc.store_scatter(tie_v, [idx], k, mask=m)

        b2, cand_off, n3, need = level(load_tie, store_row_bits, 16, n2, need, cand_off)
        b3, cand_off, n4, need = level(load_row_bits, store_tie, 8, n3, need, cand_off)
        b4, cand_off, _, _ = level(load_tie, None, 0, n4, need, cand_off)

        kt = (lax.shift_left(b1 - 128, 24) | lax.shift_left(b2, 16)
              | lax.shift_left(b3, 8) | b4)

        def pad(i, c):
            idxv = i * 16 + lane
            cur = cand_v[pl.ds(i * 16, 16)]
            cand_v[pl.ds(i * 16, 16)] = jnp.where(idxv < cand_off, cur, kt)
            return c
        lax.fori_loop(0, K // 16, pad, 0)

        def vsort_pass(i, c):
            v = cand_v[pl.ds(i * 16, 16)]
            sk, _ = plsc.sort_key_val(v, v, descending=True)
            cand_v[pl.ds(i * 16, 16)] = sk
            return c
        lax.fori_loop(0, K // 16, vsort_pass, 0)

        nvk = K // 16
        for t in range(6):
            nb = 1 << t
            if nb == 1:
                def rev1(g, c):
                    a = 2 * g + 1
                    cand_v[pl.ds(a * 16, 16)] = lax.rev(
                        cand_v[pl.ds(a * 16, 16)], (0,))
                    return c
                lax.fori_loop(0, nvk // 2, rev1, 0)
            else:
                half = nb // 2
                def revp(p, c, t=t, nb=nb, half=half):
                    g = lax.shift_right_logical(p, t - 1)
                    i = p & (half - 1)
                    base = g * 2 * nb + nb
                    a = base + i
                    b = base + nb - 1 - i
                    va = lax.rev(cand_v[pl.ds(a * 16, 16)], (0,))
                    vb = lax.rev(cand_v[pl.ds(b * 16, 16)], (0,))
                    cand_v[pl.ds(a * 16, 16)] = vb
                    cand_v[pl.ds(b * 16, 16)] = va
                    return c
                lax.fori_loop(0, (nvk // (2 * nb)) * half, revp, 0)
            for sub in range(t, -1, -1):
                s = 1 << sub
                def ce(m, c, sub=sub, s=s):
                    ia = lax.shift_left(lax.shift_right_logical(m, sub), sub + 1) | (m & (s - 1))
                    ib = ia + s
                    va = cand_v[pl.ds(ia * 16, 16)]
                    vb = cand_v[pl.ds(ib * 16, 16)]
                    cand_v[pl.ds(ia * 16, 16)] = jnp.maximum(va, vb)
                    cand_v[pl.ds(ib * 16, 16)] = jnp.minimum(va, vb)
                    return c
                lax.fori_loop(0, nvk // 2, ce, 0)
            lax.fori_loop(0, nvk, vsort_pass, 0)

        def inv_store(i, c):
            k = cand_v[pl.ds(i * 16, 16)]
            b = k ^ lax.shift_right_logical(lax.shift_right_arithmetic(k, 31), 1)
            orow_v[pl.ds(i * 16, 16)] = plsc.bitcast(b, jnp.float32)
            return c
        lax.fori_loop(0, K // 16, inv_store, 0)
        pltpu.sync_copy(orow_v, out_hbm.at[r])
        return carry

    lax.fori_loop(0, RPW, do_row, 0)


_kmax_sc = pl.kernel(
    _kmax_body,
    out_type=jax.ShapeDtypeStruct((R, K), jnp.float32),
    mesh=_mesh,
    scratch_types=_SCRATCH,
    compiler_params=pltpu.CompilerParams(needs_layout_passes=False),
)


@jax.jit
def kernel(x):
    x2 = x.reshape(R, N)
    out = _kmax_sc(x2)
    return out.reshape(x.shape[0], x.shape[1], K)


# speedup vs baseline: 22.4343x; 3.3740x over previous
import functools

import jax
import jax.numpy as jnp
from jax import lax
from jax.experimental import pallas as pl
from jax.experimental.pallas import tpu as pltpu
from jax.experimental.pallas import tpu_sc as plsc

K = 1024
N = 32768
R = 1024
NV = N // 16
NC, NS = 2, 16
NW = NC * NS
RPW = R // NW

_mesh = plsc.VectorSubcoreMesh(
    core_axis_name="c", subcore_axis_name="s", num_cores=NC, num_subcores=NS
)


_SCRATCH = [
    pltpu.VMEM((N + 16,), jnp.float32),
    pltpu.VMEM((N + 16,), jnp.int32),
    pltpu.VMEM((4096,), jnp.int32),
    pltpu.VMEM((K + 16,), jnp.int32),
    pltpu.VMEM((K,), jnp.float32),
]


def _kmax_body(x_hbm, out_hbm, row_v, tie_v, hist_v, cand_v, orow_v):
    wid = lax.axis_index("s") * NC + lax.axis_index("c")
    lane = lax.iota(jnp.int32, 16)
    ones = jnp.ones((16,), jnp.int32)
    zeros16 = jnp.zeros((16,), jnp.int32)

    def mono16(vf):
        b = plsc.bitcast(vf, jnp.int32)
        return b ^ lax.shift_right_logical(lax.shift_right_arithmetic(b, 31), 1)

    def clear_hist():
        @plsc.parallel_loop(0, 256, unroll=8)
        def _clr(i):
            hist_v[pl.ds(i * 16, 16)] = zeros16

    def find_bstar(need):
        def cond(c):
            _, acc, _ = c
            return acc < need
        def body(c):
            b, acc, _ = c
            b2 = b - 1
            cnt = jnp.sum(hist_v[pl.ds(b2 * 16, 16)])
            return (b2, acc + cnt, cnt)
        b, acc, last = lax.while_loop(
            cond, body, (jnp.int32(256), jnp.int32(0), jnp.int32(0))
        )
        return b, acc - last, last

    def do_row(j, carry):
        r = wid * RPW + j
        pltpu.sync_copy(x_hbm.at[r], row_v.at[pl.ds(0, N)])

        clear_hist()

        @plsc.parallel_loop(0, NV, unroll=8)
        def _h1(i):
            k = mono16(row_v[pl.ds(i * 16, 16)])
            b = lax.shift_right_arithmetic(k, 24) + 128
            plsc.addupdate_scatter(hist_v, [b * 16 + lane], ones)

        b1, g1, n2 = find_bstar(jnp.int32(K))

        zv = jnp.zeros((16,), jnp.int32)

        @plsc.parallel_loop(0, NV, unroll=4, carry=(zv, zv))
        def _c1(i, carry):
            cov, tov = carry
            k = mono16(row_v[pl.ds(i * 16, 16)])
            b = lax.shift_right_arithmetic(k, 24) + 128
            mgt = b > b1
            meq = b == b1
            gi = mgt.astype(jnp.int32)
            ei = meq.astype(jnp.int32)
            plsc.store_scatter(cand_v, [cov + plsc.cumsum(gi) - gi], k, mask=mgt)
            plsc.store_scatter(tie_v, [tov + plsc.cumsum(ei) - ei], k, mask=meq)
            return (cov + plsc.all_reduce_population_count(mgt),
                    tov + plsc.all_reduce_population_count(meq))
        cand_off = g1
        need = jnp.int32(K) - g1

        def level(load_fn, store_fn, shift, n, need, cand_off):
            clear_hist()
            nv = lax.shift_right_logical(n + 15, 4)

            @plsc.parallel_loop(0, nv, unroll=2)
            def _h(i):
                k = load_fn(i)
                valid = (i * 16 + lane) < n
                b = jnp.bitwise_and(lax.shift_right_logical(k, shift), 0xFF)
                plsc.addupdate_scatter(hist_v, [b * 16 + lane], ones, mask=valid)

            bs, g, n_next = find_bstar(need)
            zv = jnp.zeros((16,), jnp.int32)
            cov0 = jnp.broadcast_to(cand_off, (16,)).astype(jnp.int32)

            @plsc.parallel_loop(0, nv, unroll=2, carry=(cov0, zv))
            def _c(i, carry):
                cov, tov = carry
                k = load_fn(i)
                valid = (i * 16 + lane) < n
                b = jnp.bitwise_and(lax.shift_right_logical(k, shift), 0xFF)
                mgt = (b > bs) & valid
                gi = mgt.astype(jnp.int32)
                plsc.store_scatter(cand_v, [cov + plsc.cumsum(gi) - gi], k, mask=mgt)
                cov = cov + plsc.all_reduce_population_count(mgt)
                if store_fn is not None:
                    meq = (b == bs) & valid
                    ei = meq.astype(jnp.int32)
                    store_fn(tov + plsc.cumsum(ei) - ei, k, meq)
                    tov = tov + plsc.all_reduce_population_count(meq)
                return (cov, tov)
            return bs, cand_off + g, n_next, need - g

        def load_tie(i):
            return tie_v[pl.ds(i * 16, 16)]

        def load_row_bits(i):
            return plsc.bitcast(row_v[pl.ds(i * 16, 16)], jnp.int32)

        def store_row_bits(idx, k, m):
            plsc.store_scatter(row_v, [idx], plsc.bitcast(k, jnp.float32), mask=m)

        def store_tie(idx, k, m):
            plsc.store_scatter(tie_v, [idx], k, mask=m)

        b2, cand_off, n3, need = level(load_tie, store_row_bits, 16, n2, need, cand_off)
        b3, cand_off, n4, need = level(load_row_bits, store_tie, 8, n3, need, cand_off)
        b4, cand_off, _, _ = level(load_tie, None, 0, n4, need, cand_off)

        kt = (lax.shift_left(b1 - 128, 24) | lax.shift_left(b2, 16)
              | lax.shift_left(b3, 8) | b4)

        @plsc.parallel_loop(0, K // 16, unroll=4)
        def _pad(i):
            idxv = i * 16 + lane
            cur = cand_v[pl.ds(i * 16, 16)]
            cand_v[pl.ds(i * 16, 16)] = jnp.where(idxv < cand_off, cur, kt)

        def vsort_all():
            @plsc.parallel_loop(0, K // 16, unroll=4)
            def _vs(i):
                v = cand_v[pl.ds(i * 16, 16)]
                sk, _ = plsc.sort_key_val(v, v, descending=True)
                cand_v[pl.ds(i * 16, 16)] = sk

        vsort_all()
        nvk = K // 16
        for t in range(6):
            nb = 1 << t
            if nb == 1:
                @plsc.parallel_loop(0, nvk // 2, unroll=4)
                def _rev1(g):
                    a = 2 * g + 1
                    cand_v[pl.ds(a * 16, 16)] = lax.rev(
                        cand_v[pl.ds(a * 16, 16)], (0,))
            else:
                half = nb // 2
                @plsc.parallel_loop(0, (nvk // (2 * nb)) * half, unroll=4)
                def _revp(p, t=t, nb=nb, half=half):
                    g = lax.shift_right_logical(p, t - 1)
                    i = p & (half - 1)
                    base = g * 2 * nb + nb
                    a = base + i
                    b = base + nb - 1 - i
                    va = lax.rev(cand_v[pl.ds(a * 16, 16)], (0,))
                    vb = lax.rev(cand_v[pl.ds(b * 16, 16)], (0,))
                    cand_v[pl.ds(a * 16, 16)] = vb
                    cand_v[pl.ds(b * 16, 16)] = va
            for sub in range(t, -1, -1):
                s = 1 << sub
                @plsc.parallel_loop(0, nvk // 2, unroll=4)
                def _ce(m, sub=sub, s=s):
                    ia = lax.shift_left(lax.shift_right_logical(m, sub), sub + 1) | (m & (s - 1))
                    ib = ia + s
                    va = cand_v[pl.ds(ia * 16, 16)]
                    vb = cand_v[pl.ds(ib * 16, 16)]
                    cand_v[pl.ds(ia * 16, 16)] = jnp.maximum(va, vb)
                    cand_v[pl.ds(ib * 16, 16)] = jnp.minimum(va, vb)
            vsort_all()

        @plsc.parallel_loop(0, K // 16, unroll=4)
        def _inv(i):
            k = cand_v[pl.ds(i * 16, 16)]
            b = k ^ lax.shift_right_logical(lax.shift_right_arithmetic(k, 31), 1)
            orow_v[pl.ds(i * 16, 16)] = plsc.bitcast(b, jnp.float32)
        pltpu.sync_copy(orow_v, out_hbm.at[r])
        return carry

    lax.fori_loop(0, RPW, do_row, 0)


_kmax_sc = pl.kernel(
    _kmax_body,
    out_type=jax.ShapeDtypeStruct((R, K), jnp.float32),
    mesh=_mesh,
    scratch_types=_SCRATCH,
    compiler_params=pltpu.CompilerParams(needs_layout_passes=False),
)


@jax.jit
def kernel(x):
    x2 = x.reshape(R, N)
    out = _kmax_sc(x2)
    return out.reshape(x.shape[0], x.shape[1], K)
